# Initial kernel scaffold; baseline (speedup 1.0000x reference)
import jax
import jax.numpy as jnp
from jax.experimental import pallas as pl


def kernel(item_seq, item_seq_len):
    raise NotImplementedError("write your pallas kernel here")


# speedup vs baseline: 2.6301x; 2.6301x over previous
import jax
import jax.numpy as jnp
from jax import lax
from jax.experimental import pallas as pl
from jax.experimental.pallas import tpu as pltpu

_CROP_RATIO = 0.6
_MIN_LENGTH = 3
_MASK_RATIO = 0.3
_REORDER_RATIO = 0.5
_MIN_W = 2
_MAX_W = 5
_ROWS = 256


def _aug_body(seq_ref, len_ref, uc_ref, sc_ref, uw_ref, us_ref, r5_ref,
              ua_ref, crop_ref, cl_ref, mask_ref, reord_ref):
    seq = seq_ref[...]
    lens = len_ref[...]
    R, L = seq.shape
    pos = lax.broadcasted_iota(jnp.int32, (R, L), 1)
    zero = jnp.zeros_like(seq)

    lens_f = lens.astype(jnp.float32)
    crop_len = jnp.maximum(_MIN_LENGTH, (lens_f * _CROP_RATIO).astype(jnp.int32))
    crop_len = jnp.minimum(crop_len, lens)
    max_start = jnp.maximum(lens - crop_len + 1, 1)
    start = jnp.minimum(
        jnp.floor(uc_ref[...] * max_start.astype(jnp.float32)).astype(jnp.int32),
        max_start - 1)
    shifted = seq
    for b in range(8):
        sh = 1 << b
        if sh >= L:
            break
        moved = jnp.concatenate(
            [shifted[:, sh:], jnp.zeros((R, sh), jnp.int32)], axis=1)
        shifted = jnp.where(((start >> b) & 1) == 1, moved, shifted)
    out_c = jnp.where(pos < crop_len, shifted, zero)
    apply_c = lens > _MIN_LENGTH
    crop_ref[...] = jnp.where(apply_c, out_c, seq)
    cl_ref[...] = jnp.where(apply_c, crop_len, lens)

    valid = (seq != 0) & (pos < lens)
    vi = valid.astype(jnp.int32)
    n_valid = jnp.sum(vi, axis=1, keepdims=True)
    num_to_mask = jnp.minimum(
        jnp.maximum(1, (n_valid.astype(jnp.float32) * _MASK_RATIO).astype(jnp.int32)),
        n_valid)
    sc = jnp.where(valid, sc_ref[...], 2.0)
    kp1 = jnp.clip(num_to_mask - 1, 0, L - 1) + 1
    lo = jnp.zeros_like(lens)
    hi = jnp.full_like(lens, 1 << 23)
    for _ in range(24):
        mid = (lo + hi) >> 1
        t = lax.bitcast_convert_type(mid + 0x3F800000, jnp.float32) - 1.0
        cnt = jnp.sum((sc <= t).astype(jnp.int32), axis=1, keepdims=True)
        ge = cnt >= kp1
        hi = jnp.where(ge, mid, hi)
        lo = jnp.where(ge, lo, mid + 1)
    thresh = lax.bitcast_convert_type(hi + 0x3F800000, jnp.float32) - 1.0
    apply_m = (lens > 1) & (n_valid > 0)
    to_mask = valid & (sc <= thresh)
    mask_ref[...] = jnp.where(apply_m & to_mask, zero, seq)

    max_possible = jnp.minimum(n_valid, _MAX_W)
    w = _MIN_W + jnp.floor(
        uw_ref[...] * jnp.maximum(max_possible - _MIN_W + 1, 1).astype(jnp.float32)
    ).astype(jnp.int32)
    w = jnp.clip(w, _MIN_W, jnp.maximum(max_possible, _MIN_W))
    max_start2 = jnp.maximum(n_valid - w + 1, 1)
    s = jnp.minimum(
        jnp.floor(us_ref[...] * max_start2.astype(jnp.float32)).astype(jnp.int32),
        max_start2 - 1)
    applied = (ua_ref[...] <= _REORDER_RATIO) & (lens > _MIN_W) & (n_valid >= _MIN_W)

    c = vi
    sh = 1
    while sh < L:
        moved = jnp.concatenate(
            [jnp.zeros((R, sh), jnp.int32), c[:, :L - sh]], axis=1)
        c = c + moved
        sh <<= 1
    excl = c - vi

    posk, valk = [], []
    for k in range(_MAX_W):
        hit = valid & (excl == s + k)
        posk.append(jnp.sum(jnp.where(hit, pos, 0), axis=1, keepdims=True))
        valk.append(jnp.sum(jnp.where(hit, seq, 0), axis=1, keepdims=True))

    r5 = r5_ref[...]
    rk = [jnp.where(k < w, r5[:, k:k + 1], 2.0) for k in range(_MAX_W)]
    ranks = []
    for i in range(_MAX_W):
        acc = jnp.zeros_like(lens)
        for j in range(_MAX_W):
            if j == i:
                continue
            cmp = (rk[j] <= rk[i]) if j < i else (rk[j] < rk[i])
            acc = acc + cmp.astype(jnp.int32)
        ranks.append(acc)

    out_r = seq
    for p in range(_MAX_W):
        vsrc = jnp.zeros_like(lens)
        for i in range(_MAX_W):
            vsrc = vsrc + jnp.where(ranks[i] == p, valk[i], 0)
        cond = applied & (p < w) & (pos == posk[p])
        out_r = jnp.where(cond, vsrc, out_r)
    reord_ref[...] = out_r


def kernel(item_seq, item_seq_len):
    B, L = item_seq.shape
    R = _ROWS

    key = jax.random.key(42)
    kc, km, kr = jax.random.split(key, 3)
    u_crop = jax.random.uniform(kc, (B,))
    scores = jax.random.uniform(km, (B, L))
    k1, k2, k3, k4 = jax.random.split(kr, 4)
    u_w = jax.random.uniform(k1, (B,))
    u_s = jax.random.uniform(k2, (B,))
    r = jax.random.uniform(k3, (B, _MAX_W))
    u_apply = jax.random.uniform(k4, (B,))
    r5 = jnp.pad(r, ((0, 0), (0, 8 - _MAX_W)), constant_values=2.0)

    lens = item_seq_len.astype(jnp.int32).reshape(B, 1)
    col = lambda x: x.reshape(B, 1)

    row_spec = pl.BlockSpec((R, L), lambda i: (i, 0))
    col_spec = pl.BlockSpec((R, 1), lambda i: (i, 0))
    r5_spec = pl.BlockSpec((R, 8), lambda i: (i, 0))

    cs, cl, ms, rs = pl.pallas_call(
        _aug_body,
        grid=(B // R,),
        in_specs=[row_spec, col_spec, col_spec, row_spec, col_spec,
                  col_spec, r5_spec, col_spec],
        out_specs=[row_spec, col_spec, row_spec, row_spec],
        out_shape=[
            jax.ShapeDtypeStruct((B, L), jnp.int32),
            jax.ShapeDtypeStruct((B, 1), jnp.int32),
            jax.ShapeDtypeStruct((B, L), jnp.int32),
            jax.ShapeDtypeStruct((B, L), jnp.int32),
        ],
        compiler_params=pltpu.CompilerParams(
            dimension_semantics=("parallel",)),
    )(item_seq, lens, col(u_crop), scores, col(u_w), col(u_s), r5,
      col(u_apply))

    ml = item_seq_len.astype(jnp.int32)
    return cs, cl.reshape(B), ms, ml, rs, ml


# speedup vs baseline: 9.0773x; 3.4512x over previous
import jax
import jax.numpy as jnp
from jax import lax
from jax.experimental import pallas as pl
from jax.experimental.pallas import tpu as pltpu

_CROP_RATIO = 0.6
_MIN_LENGTH = 3
_MASK_RATIO = 0.3
_REORDER_RATIO = 0.5
_MIN_W = 2
_MAX_W = 5
_COLS = 512


def _aug_body(seq_ref, len_ref, uc_ref, sc_ref, uw_ref, us_ref, r8_ref,
              ua_ref, crop_ref, cl_ref, mask_ref, reord_ref):
    seq = seq_ref[...]
    lens = len_ref[...]
    L, C = seq.shape
    pos = lax.broadcasted_iota(jnp.int32, (L, C), 0)
    zero = jnp.zeros_like(seq)

    lens_f = lens.astype(jnp.float32)
    crop_len = jnp.maximum(_MIN_LENGTH, (lens_f * _CROP_RATIO).astype(jnp.int32))
    crop_len = jnp.minimum(crop_len, lens)
    max_start = jnp.maximum(lens - crop_len + 1, 1)
    start = jnp.minimum(
        jnp.floor(uc_ref[...] * max_start.astype(jnp.float32)).astype(jnp.int32),
        max_start - 1)
    shifted = seq
    for b in range(8):
        sh = 1 << b
        if sh >= L:
            break
        moved = jnp.concatenate(
            [shifted[sh:, :], jnp.zeros((sh, C), jnp.int32)], axis=0)
        shifted = jnp.where(((start >> b) & 1) == 1, moved, shifted)
    out_c = jnp.where(pos < crop_len, shifted, zero)
    apply_c = lens > _MIN_LENGTH
    crop_ref[...] = jnp.where(apply_c, out_c, seq)
    cl_ref[...] = jnp.where(apply_c, crop_len, lens)

    valid = (seq != 0) & (pos < lens)
    vi = valid.astype(jnp.int32)
    n_valid = jnp.sum(vi, axis=0, keepdims=True)
    num_to_mask = jnp.minimum(
        jnp.maximum(1, (n_valid.astype(jnp.float32) * _MASK_RATIO).astype(jnp.int32)),
        n_valid)
    sc = jnp.where(valid, sc_ref[...], 2.0)
    kp1 = jnp.clip(num_to_mask - 1, 0, L - 1) + 1
    lo = jnp.zeros_like(lens)
    hi = jnp.full_like(lens, 1 << 23)
    for _ in range(24):
        mid = (lo + hi) >> 1
        t = lax.bitcast_convert_type(mid + 0x3F800000, jnp.float32) - 1.0
        cnt = jnp.sum((sc <= t).astype(jnp.int32), axis=0, keepdims=True)
        ge = cnt >= kp1
        hi = jnp.where(ge, mid, hi)
        lo = jnp.where(ge, lo, mid + 1)
    thresh = lax.bitcast_convert_type(hi + 0x3F800000, jnp.float32) - 1.0
    apply_m = (lens > 1) & (n_valid > 0)
    to_mask = valid & (sc <= thresh)
    mask_ref[...] = jnp.where(apply_m & to_mask, zero, seq)

    max_possible = jnp.minimum(n_valid, _MAX_W)
    w = _MIN_W + jnp.floor(
        uw_ref[...] * jnp.maximum(max_possible - _MIN_W + 1, 1).astype(jnp.float32)
    ).astype(jnp.int32)
    w = jnp.clip(w, _MIN_W, jnp.maximum(max_possible, _MIN_W))
    max_start2 = jnp.maximum(n_valid - w + 1, 1)
    s = jnp.minimum(
        jnp.floor(us_ref[...] * max_start2.astype(jnp.float32)).astype(jnp.int32),
        max_start2 - 1)
    applied = (ua_ref[...] <= _REORDER_RATIO) & (lens > _MIN_W) & (n_valid >= _MIN_W)

    c = vi
    sh = 1
    while sh < L:
        moved = jnp.concatenate(
            [jnp.zeros((sh, C), jnp.int32), c[:L - sh, :]], axis=0)
        c = c + moved
        sh <<= 1
    excl = c - vi

    posk, valk = [], []
    for k in range(_MAX_W):
        hit = valid & (excl == s + k)
        posk.append(jnp.sum(jnp.where(hit, pos, 0), axis=0, keepdims=True))
        valk.append(jnp.sum(jnp.where(hit, seq, 0), axis=0, keepdims=True))

    r8 = r8_ref[...]
    rk = [jnp.where(k < w, r8[k:k + 1, :], 2.0) for k in range(_MAX_W)]
    ranks = []
    for i in range(_MAX_W):
        acc = jnp.zeros_like(lens)
        for j in range(_MAX_W):
            if j == i:
                continue
            cmp = (rk[j] <= rk[i]) if j < i else (rk[j] < rk[i])
            acc = acc + cmp.astype(jnp.int32)
        ranks.append(acc)

    out_r = seq
    for p in range(_MAX_W):
        vsrc = jnp.zeros_like(lens)
        for i in range(_MAX_W):
            vsrc = vsrc + jnp.where(ranks[i] == p, valk[i], 0)
        cond = applied & (p < w) & (pos == posk[p])
        out_r = jnp.where(cond, vsrc, out_r)
    reord_ref[...] = out_r


def kernel(item_seq, item_seq_len):
    B, L = item_seq.shape
    C = _COLS

    key = jax.random.key(42)
    kc, km, kr = jax.random.split(key, 3)
    u_crop = jax.random.uniform(kc, (B,))
    scores = jax.random.uniform(km, (B, L))
    k1, k2, k3, k4 = jax.random.split(kr, 4)
    u_w = jax.random.uniform(k1, (B,))
    u_s = jax.random.uniform(k2, (B,))
    r = jax.random.uniform(k3, (B, _MAX_W))
    u_apply = jax.random.uniform(k4, (B,))
    r8 = jnp.pad(r.T, ((0, 8 - _MAX_W), (0, 0)), constant_values=2.0)

    seq_t = item_seq.T
    scores_t = scores.T
    lens = item_seq_len.astype(jnp.int32).reshape(1, B)
    row = lambda x: x.reshape(1, B)

    big_spec = pl.BlockSpec((L, C), lambda i: (0, i))
    one_spec = pl.BlockSpec((1, C), lambda i: (0, i))
    r8_spec = pl.BlockSpec((8, C), lambda i: (0, i))

    cs, cl, ms, rs = pl.pallas_call(
        _aug_body,
        grid=(B // C,),
        in_specs=[big_spec, one_spec, one_spec, big_spec, one_spec,
                  one_spec, r8_spec, one_spec],
        out_specs=[big_spec, one_spec, big_spec, big_spec],
        out_shape=[
            jax.ShapeDtypeStruct((L, B), jnp.int32),
            jax.ShapeDtypeStruct((1, B), jnp.int32),
            jax.ShapeDtypeStruct((L, B), jnp.int32),
            jax.ShapeDtypeStruct((L, B), jnp.int32),
        ],
        compiler_params=pltpu.CompilerParams(
            dimension_semantics=("parallel",)),
    )(seq_t, lens, row(u_crop), scores_t, row(u_w), row(u_s), r8,
      row(u_apply))

    ml = item_seq_len.astype(jnp.int32)
    return cs.T, cl.reshape(B), ms.T, ml, rs.T, ml
